# Initial kernel scaffold; baseline (speedup 1.0000x reference)
import jax
import jax.numpy as jnp
from jax.experimental import pallas as pl


def kernel(x, edge_index, W1, b1, W2, b2, prelu_a, bn_g, bn_b, pW, pb, pbn_g, pbn_b, prelu_a2):
    raise NotImplementedError("write your pallas kernel here")


# speedup vs baseline: 13.4850x; 13.4850x over previous
import functools

import jax
import jax.numpy as jnp
from jax import lax
from jax.experimental import pallas as pl
from jax.experimental.pallas import tpu as pltpu
from jax.experimental.pallas import tpu_sc as plsc

NC = 2
NS = 16
NW = NC * NS
CHUNK = 80

_HIGH = jax.lax.Precision.HIGHEST


def _hist_body(npad, nchunk, col3_hbm, out_hbm, idx_v, ones_v, zb_v, acc_sh):
    cid = lax.axis_index("c")
    sid = lax.axis_index("s")
    wid = cid * NS + sid
    zrow = npad // NS
    zeros16 = jnp.zeros((16,), jnp.float32)
    ones16 = jnp.full((16,), 1.0, jnp.float32)
    for k in range(CHUNK // 16):
        ones_v[pl.ds(k * 16, 16)] = ones16

    def zfill(j, carry):
        zb_v[pl.ds(j * 16, 16)] = zeros16
        return carry

    lax.fori_loop(0, zrow // 16, zfill, 0)
    pltpu.sync_copy(zb_v, acc_sh.at[pl.ds(sid * zrow, zrow)])
    plsc.subcore_barrier()
    pltpu.sync_copy(col3_hbm.at[wid], idx_v)

    def step(i, carry):
        pltpu.sync_copy(ones_v, acc_sh.at[idx_v.at[i]], add=True)
        return carry

    lax.fori_loop(0, nchunk, step, 0)
    plsc.subcore_barrier()
    pltpu.sync_copy(acc_sh.at[pl.ds(sid * zrow, zrow)],
                    out_hbm.at[pl.ds(wid * zrow, zrow)])


def _make_hist(e, npad):
    nchunk = e // (NW * CHUNK)
    mesh = plsc.VectorSubcoreMesh(core_axis_name="c", subcore_axis_name="s")
    return pl.kernel(
        functools.partial(_hist_body, npad, nchunk),
        out_type=jax.ShapeDtypeStruct((NC * npad,), jnp.float32),
        mesh=mesh,
        scratch_types=[
            pltpu.VMEM((nchunk, CHUNK), jnp.int32),
            pltpu.VMEM((CHUNK,), jnp.float32),
            pltpu.VMEM((npad // NS,), jnp.float32),
            pltpu.VMEM_SHARED((npad,), jnp.float32),
        ],
    )


def _scatter_body(n, hh, nchunk, g_hbm, row4_hbm, col3_hbm, out_hbm,
                  idxr_v, idxc_v, rows_v, zb_v, acc_sh, sem):
    cid = lax.axis_index("c")
    sid = lax.axis_index("s")
    rows_per_tile = n // NS
    zrows = zb_v.shape[0]
    zeros16 = jnp.zeros((16,), jnp.float32)

    def zfill(j, carry):
        for k in range(hh // 16):
            zb_v[j, pl.ds(k * 16, 16)] = zeros16
        return carry

    lax.fori_loop(0, zrows, zfill, 0)
    for t in range(rows_per_tile // zrows):
        pltpu.sync_copy(zb_v, acc_sh.at[pl.ds(sid * rows_per_tile + t * zrows, zrows)])
    plsc.subcore_barrier()

    pltpu.sync_copy(row4_hbm.at[cid, sid], idxr_v)
    pltpu.sync_copy(col3_hbm.at[sid], idxc_v)

    def step(i, carry):
        pltpu.async_copy(g_hbm.at[idxr_v.at[i]], rows_v, sem).wait()
        pltpu.sync_copy(rows_v, acc_sh.at[idxc_v.at[i]], add=True)
        return carry

    lax.fori_loop(0, nchunk, step, 0)
    plsc.subcore_barrier()
    pltpu.sync_copy(acc_sh.at[pl.ds(sid * rows_per_tile, rows_per_tile)],
                    out_hbm.at[cid, sid])


def _make_scatter(n, hh, e):
    nchunk = e // (NS * CHUNK)
    mesh = plsc.VectorSubcoreMesh(core_axis_name="c", subcore_axis_name="s")
    return pl.kernel(
        functools.partial(_scatter_body, n, hh, nchunk),
        out_type=jax.ShapeDtypeStruct((NC, NS, n // NS, hh), jnp.float32),
        mesh=mesh,
        scratch_types=[
            pltpu.VMEM((nchunk, CHUNK), jnp.int32),
            pltpu.VMEM((nchunk, CHUNK), jnp.int32),
            pltpu.VMEM((CHUNK, hh), jnp.float32),
            pltpu.VMEM((125, hh), jnp.float32),
            pltpu.VMEM_SHARED((n, hh), jnp.float32),
            pltpu.SemaphoreType.DMA,
        ],
        compiler_params=pltpu.CompilerParams(use_tc_tiling_on_sc=False),
    )


def _tc_g1(x_ref, w1_ref, dis_ref, g1_ref):
    hm = jnp.dot(x_ref[...], w1_ref[...], preferred_element_type=jnp.float32,
                 precision=_HIGH)
    g1_ref[...] = hm * dis_ref[...]


def _tc_mid(s_ref, g1_ref, dis_ref, b1_ref, a1_ref, w2_ref, g2_ref):
    o = (s_ref[...] + g1_ref[...]) * dis_ref[...] + b1_ref[...]
    a = a1_ref[0, 0]
    z1 = jnp.where(o >= 0, o, a * o)
    h2 = jnp.dot(z1, w2_ref[...], preferred_element_type=jnp.float32,
                 precision=_HIGH)
    g2_ref[...] = h2 * dis_ref[...]


def _tc_fin(s_ref, g2_ref, dis_ref, b2_ref, a_ref, bng_ref, bnb_ref,
            pw_ref, pb_ref, pbng_ref, pbnb_ref, a2_ref, z_ref, p_ref):
    o = (s_ref[...] + g2_ref[...]) * dis_ref[...] + b2_ref[...]
    a = a_ref[0, 0]
    z2 = jnp.where(o >= 0, o, a * o)
    m = jnp.mean(z2, axis=0, keepdims=True)
    v = jnp.mean((z2 - m) ** 2, axis=0, keepdims=True)
    z = (z2 - m) * lax.rsqrt(v + 1e-5) * bng_ref[...] + bnb_ref[...]
    q = jnp.dot(z, pw_ref[...], preferred_element_type=jnp.float32,
                precision=_HIGH) + pb_ref[...]
    m2 = jnp.mean(q, axis=0, keepdims=True)
    v2 = jnp.mean((q - m2) ** 2, axis=0, keepdims=True)
    pz = (q - m2) * lax.rsqrt(v2 + 1e-5) * pbng_ref[...] + pbnb_ref[...]
    a2 = a2_ref[0, 0]
    z_ref[...] = z
    p_ref[...] = jnp.where(pz >= 0, pz, a2 * pz)


def _to_split(g, n, hh):
    return g.reshape(n, 2, hh).transpose(1, 0, 2).reshape(2 * n, hh)


def _from_split(sp, n, hh):
    s3 = sp.reshape(NC, n, hh)
    return jnp.concatenate([s3[0], s3[1]], axis=1)


def kernel(x, edge_index, W1, b1, W2, b2, prelu_a, bn_g, bn_b, pW, pb,
           pbn_g, pbn_b, prelu_a2):
    n, d = x.shape
    h = W1.shape[1]
    hh = h // 2
    e = edge_index.shape[1]
    npad = ((n + 16 * NS - 1) // (16 * NS)) * (16 * NS)

    row = edge_index[0]
    col = edge_index[1]
    nch_h = e // (NW * CHUNK)
    col3h = col.reshape(NW, nch_h, CHUNK)
    nch_s = e // (NS * CHUNK)
    col3 = col.reshape(NS, nch_s, CHUNK)
    row4 = jnp.stack([row, row + n]).reshape(NC, NS, nch_s, CHUNK)

    degf = _make_hist(e, npad)(col3h)
    degp = degf.reshape(NC, npad)
    dis = lax.rsqrt(degp[0, :n] + degp[1, :n] + 1.0)[:, None]

    b1r = b1.reshape(1, h)
    b2r = b2.reshape(1, h)
    a1r = prelu_a.reshape(1, 1)
    a2r = prelu_a2.reshape(1, 1)

    g1 = pl.pallas_call(
        _tc_g1, out_shape=jax.ShapeDtypeStruct((n, h), jnp.float32),
    )(x, W1, dis)

    scat = _make_scatter(n, hh, e)
    s1 = _from_split(scat(_to_split(g1, n, hh), row4, col3), n, hh)

    g2 = pl.pallas_call(
        _tc_mid, out_shape=jax.ShapeDtypeStruct((n, h), jnp.float32),
    )(s1, g1, dis, b1r, a1r, W2)

    s2 = _from_split(scat(_to_split(g2, n, hh), row4, col3), n, hh)

    z, p = pl.pallas_call(
        _tc_fin,
        out_shape=[jax.ShapeDtypeStruct((n, h), jnp.float32),
                   jax.ShapeDtypeStruct((n, h), jnp.float32)],
    )(s2, g2, dis, b2r, a1r, bn_g.reshape(1, h), bn_b.reshape(1, h),
      pW, pb.reshape(1, h), pbn_g.reshape(1, h), pbn_b.reshape(1, h), a2r)

    return (z, p)


# speedup vs baseline: 17.6805x; 1.3111x over previous
import functools

import jax
import jax.numpy as jnp
from jax import lax
from jax.experimental import pallas as pl
from jax.experimental.pallas import tpu as pltpu
from jax.experimental.pallas import tpu_sc as plsc

NC = 2
NS = 16
NW = NC * NS
CHUNK = 80

_HIGH = jax.lax.Precision.HIGHEST


def _hist_body(npad, nchunk, col3_hbm, out_hbm, idx_v, ones_v, zb_v, acc_sh):
    cid = lax.axis_index("c")
    sid = lax.axis_index("s")
    wid = cid * NS + sid
    zrow = npad // NS
    zeros16 = jnp.zeros((16,), jnp.float32)
    ones16 = jnp.full((16,), 1.0, jnp.float32)
    for k in range(CHUNK // 16):
        ones_v[pl.ds(k * 16, 16)] = ones16

    def zfill(j, carry):
        zb_v[pl.ds(j * 16, 16)] = zeros16
        return carry

    lax.fori_loop(0, zrow // 16, zfill, 0)
    pltpu.sync_copy(zb_v, acc_sh.at[pl.ds(sid * zrow, zrow)])
    plsc.subcore_barrier()
    pltpu.sync_copy(col3_hbm.at[wid], idx_v)

    def step(i, carry):
        pltpu.sync_copy(ones_v, acc_sh.at[idx_v.at[i]], add=True)
        return carry

    lax.fori_loop(0, nchunk, step, 0)
    plsc.subcore_barrier()
    pltpu.sync_copy(acc_sh.at[pl.ds(sid * zrow, zrow)],
                    out_hbm.at[pl.ds(wid * zrow, zrow)])


def _make_hist(e, npad):
    nchunk = e // (NW * CHUNK)
    mesh = plsc.VectorSubcoreMesh(core_axis_name="c", subcore_axis_name="s")
    return pl.kernel(
        functools.partial(_hist_body, npad, nchunk),
        out_type=jax.ShapeDtypeStruct((NC * npad,), jnp.float32),
        mesh=mesh,
        scratch_types=[
            pltpu.VMEM((nchunk, CHUNK), jnp.int32),
            pltpu.VMEM((CHUNK,), jnp.float32),
            pltpu.VMEM((npad // NS,), jnp.float32),
            pltpu.VMEM_SHARED((npad,), jnp.float32),
        ],
    )


def _scatter_body(n, hh, nchunk, g_hbm, row4_hbm, col3_hbm, out_hbm,
                  idxr_v, idxc_v, rows_a, rows_b, zb_v, acc_sh, sem, sem2):
    cid = lax.axis_index("c")
    sid = lax.axis_index("s")
    rows_per_tile = n // NS
    zrows = zb_v.shape[0]
    zeros16 = jnp.zeros((16,), jnp.float32)

    def zfill(j, carry):
        for k in range(hh // 16):
            zb_v[j, pl.ds(k * 16, 16)] = zeros16
        return carry

    lax.fori_loop(0, zrows, zfill, 0)
    for t in range(rows_per_tile // zrows):
        pltpu.sync_copy(zb_v, acc_sh.at[pl.ds(sid * rows_per_tile + t * zrows, zrows)])
    plsc.subcore_barrier()

    pltpu.sync_copy(row4_hbm.at[cid, sid], idxr_v)
    pltpu.sync_copy(col3_hbm.at[sid], idxc_v)

    def pair(j, carry):
        i = j * 2
        d0 = pltpu.async_copy(g_hbm.at[idxr_v.at[i]], rows_a, sem)
        d1 = pltpu.async_copy(g_hbm.at[idxr_v.at[i + 1]], rows_b, sem2)
        d0.wait()
        pltpu.sync_copy(rows_a, acc_sh.at[idxc_v.at[i]], add=True)
        d1.wait()
        pltpu.sync_copy(rows_b, acc_sh.at[idxc_v.at[i + 1]], add=True)
        return carry

    lax.fori_loop(0, nchunk // 2, pair, 0)
    plsc.subcore_barrier()
    pltpu.sync_copy(acc_sh.at[pl.ds(sid * rows_per_tile, rows_per_tile)],
                    out_hbm.at[cid, sid])


def _make_scatter(n, hh, e):
    nchunk = e // (NS * CHUNK)
    mesh = plsc.VectorSubcoreMesh(core_axis_name="c", subcore_axis_name="s")
    return pl.kernel(
        functools.partial(_scatter_body, n, hh, nchunk),
        out_type=jax.ShapeDtypeStruct((NC, NS, n // NS, hh), jnp.float32),
        mesh=mesh,
        scratch_types=[
            pltpu.VMEM((nchunk, CHUNK), jnp.int32),
            pltpu.VMEM((nchunk, CHUNK), jnp.int32),
            pltpu.VMEM((CHUNK, hh), jnp.float32),
            pltpu.VMEM((CHUNK, hh), jnp.float32),
            pltpu.VMEM((125, hh), jnp.float32),
            pltpu.VMEM_SHARED((n, hh), jnp.float32),
            pltpu.SemaphoreType.DMA,
            pltpu.SemaphoreType.DMA,
        ],
        compiler_params=pltpu.CompilerParams(use_tc_tiling_on_sc=False),
    )


def _tc_g1(x_ref, w1_ref, dis_ref, g1_ref):
    hm = jnp.dot(x_ref[...], w1_ref[...], preferred_element_type=jnp.float32,
                 precision=_HIGH)
    g1_ref[...] = hm * dis_ref[...]


def _tc_mid(s_ref, g1_ref, dis_ref, b1_ref, a1_ref, w2_ref, g2_ref):
    o = (s_ref[...] + g1_ref[...]) * dis_ref[...] + b1_ref[...]
    a = a1_ref[0, 0]
    z1 = jnp.where(o >= 0, o, a * o)
    h2 = jnp.dot(z1, w2_ref[...], preferred_element_type=jnp.float32,
                 precision=_HIGH)
    g2_ref[...] = h2 * dis_ref[...]


def _tc_fin(s_ref, g2_ref, dis_ref, b2_ref, a_ref, bng_ref, bnb_ref,
            pw_ref, pb_ref, pbng_ref, pbnb_ref, a2_ref, z_ref, p_ref):
    o = (s_ref[...] + g2_ref[...]) * dis_ref[...] + b2_ref[...]
    a = a_ref[0, 0]
    z2 = jnp.where(o >= 0, o, a * o)
    m = jnp.mean(z2, axis=0, keepdims=True)
    v = jnp.mean((z2 - m) ** 2, axis=0, keepdims=True)
    z = (z2 - m) * lax.rsqrt(v + 1e-5) * bng_ref[...] + bnb_ref[...]
    q = jnp.dot(z, pw_ref[...], preferred_element_type=jnp.float32,
                precision=_HIGH) + pb_ref[...]
    m2 = jnp.mean(q, axis=0, keepdims=True)
    v2 = jnp.mean((q - m2) ** 2, axis=0, keepdims=True)
    pz = (q - m2) * lax.rsqrt(v2 + 1e-5) * pbng_ref[...] + pbnb_ref[...]
    a2 = a2_ref[0, 0]
    z_ref[...] = z
    p_ref[...] = jnp.where(pz >= 0, pz, a2 * pz)


def _to_split(g, n, hh):
    return g.reshape(n, 2, hh).transpose(1, 0, 2).reshape(2 * n, hh)


def _from_split(sp, n, hh):
    s3 = sp.reshape(NC, n, hh)
    return jnp.concatenate([s3[0], s3[1]], axis=1)


def kernel(x, edge_index, W1, b1, W2, b2, prelu_a, bn_g, bn_b, pW, pb,
           pbn_g, pbn_b, prelu_a2):
    n, d = x.shape
    h = W1.shape[1]
    hh = h // 2
    e = edge_index.shape[1]
    npad = ((n + 16 * NS - 1) // (16 * NS)) * (16 * NS)

    row = edge_index[0]
    col = edge_index[1]
    nch_h = e // (NW * CHUNK)
    col3h = col.reshape(NW, nch_h, CHUNK)
    nch_s = e // (NS * CHUNK)
    col3 = col.reshape(NS, nch_s, CHUNK)
    row4 = jnp.stack([row, row + n]).reshape(NC, NS, nch_s, CHUNK)

    degf = _make_hist(e, npad)(col3h)
    degp = degf.reshape(NC, npad)
    dis = lax.rsqrt(degp[0, :n] + degp[1, :n] + 1.0)[:, None]

    b1r = b1.reshape(1, h)
    b2r = b2.reshape(1, h)
    a1r = prelu_a.reshape(1, 1)
    a2r = prelu_a2.reshape(1, 1)

    g1 = pl.pallas_call(
        _tc_g1, out_shape=jax.ShapeDtypeStruct((n, h), jnp.float32),
    )(x, W1, dis)

    scat = _make_scatter(n, hh, e)
    s1 = _from_split(scat(_to_split(g1, n, hh), row4, col3), n, hh)

    g2 = pl.pallas_call(
        _tc_mid, out_shape=jax.ShapeDtypeStruct((n, h), jnp.float32),
    )(s1, g1, dis, b1r, a1r, W2)

    s2 = _from_split(scat(_to_split(g2, n, hh), row4, col3), n, hh)

    z, p = pl.pallas_call(
        _tc_fin,
        out_shape=[jax.ShapeDtypeStruct((n, h), jnp.float32),
                   jax.ShapeDtypeStruct((n, h), jnp.float32)],
    )(s2, g2, dis, b2r, a1r, bn_g.reshape(1, h), bn_b.reshape(1, h),
      pW, pb.reshape(1, h), pbn_g.reshape(1, h), pbn_b.reshape(1, h), a2r)

    return (z, p)
